# baseline (speedup 1.0000x reference)
import functools

import jax
import jax.numpy as jnp
from jax import lax
from jax.experimental import pallas as pl
from jax.experimental.pallas import tpu as pltpu
from jax.experimental.pallas import tpu_sc as plsc

_D = 16
_L = 16
_NC = 2
_NS = 16
_NW = _NC * _NS
_CHUNK = 128
_K = 4


def _gather(idx, lines):
    mesh = plsc.VectorSubcoreMesh(core_axis_name="c", subcore_axis_name="s")
    n_per_w = _K * _CHUNK

    @functools.partial(
        pl.kernel,
        out_type=jax.ShapeDtypeStruct((_NW, 64, 128), jnp.float32),
        mesh=mesh,
        scratch_types=[
            pltpu.VMEM((n_per_w,), jnp.int32),
            pltpu.VMEM((_K, _CHUNK), jnp.int32),
            pltpu.VMEM((_K, _CHUNK, 128), jnp.float32),
            pltpu.VMEM((64, 128), jnp.float32),
            pltpu.SemaphoreType.DMA,
        ],
        compiler_params=pltpu.CompilerParams(needs_layout_passes=False),
    )
    def body(idx_hbm, tbl_hbm, out_hbm, u_v, q_v, big_v, out_v, sem):
        wid = lax.axis_index("s") * _NC + lax.axis_index("c")
        pltpu.sync_copy(idx_hbm.at[pl.ds(wid * n_per_w, n_per_w)], u_v)
        iota = lax.iota(jnp.int32, _L)
        for j in range(_K):
            for g in range(_CHUNK // _L):
                u = u_v[pl.ds(j * _CHUNK + g * _L, _L)]
                q_v[j, pl.ds(g * _L, _L)] = u >> 3
        copies = [
            pltpu.async_copy(tbl_hbm.at[q_v.at[j]], big_v.at[j], sem)
            for j in range(_K)
        ]
        for j, c in enumerate(copies):
            c.wait()
            jv = jnp.full((_L,), j, jnp.int32)
            for g in range(_CHUNK // _L):
                line = g * _L + iota
                u = u_v[pl.ds(j * _CHUNK + g * _L, _L)]
                s16 = (u & 7) * _D
                f0 = (j * 2048 + g * 256) + _D * iota
                for d in range(_D):
                    vals = plsc.load_gather(big_v, [jv, line, s16 + d])
                    flat = f0 + d
                    plsc.store_scatter(out_v, [flat >> 7, flat & 127], vals)
        pltpu.sync_copy(out_v, out_hbm.at[wid])

    return body(idx, lines)


def kernel(user, user_table, item_table):
    batch = user.shape[0]
    lines = user_table.reshape(user_table.shape[0] // 8, 128)
    rows = _gather(user, lines)
    return (rows.reshape(batch, _D), item_table)


# speedup vs baseline: 3.5706x; 3.5706x over previous
import functools

import jax
import jax.numpy as jnp
from jax import lax
from jax.experimental import pallas as pl
from jax.experimental.pallas import tpu as pltpu
from jax.experimental.pallas import tpu_sc as plsc

_D = 16
_L = 16
_NC = 2
_NS = 16
_NW = _NC * _NS
_BPW = 512
_G = 16
_NB = _BPW // _G


def _gather(idx, tbl_t):
    mesh = plsc.VectorSubcoreMesh(core_axis_name="c", subcore_axis_name="s")

    @functools.partial(
        pl.kernel,
        out_type=jax.ShapeDtypeStruct((_NW, 64, 128), jnp.float32),
        mesh=mesh,
        scratch_types=[
            pltpu.VMEM((_BPW,), jnp.int32),
            pltpu.VMEM((_G, _D, 128), jnp.float32),
            pltpu.VMEM((64, 128), jnp.float32),
            pltpu.SemaphoreType.DMA,
        ],
        compiler_params=pltpu.CompilerParams(needs_layout_passes=False),
    )
    def body(idx_hbm, tbl_hbm, out_hbm, u_v, col_v, out_v, sem):
        wid = lax.axis_index("s") * _NC + lax.axis_index("c")
        pltpu.sync_copy(idx_hbm.at[pl.ds(wid * _BPW, _BPW)], u_v)
        iota = lax.iota(jnp.int32, _L)

        def batch(b, carry):
            uvec = u_v[pl.ds(b * _G, _G)]
            copies = []
            for jj in range(_G):
                c0 = pl.multiple_of((uvec[jj] >> 7) * 128, 128)
                copies.append(pltpu.async_copy(
                    tbl_hbm.at[iota, pl.ds(c0, 128)], col_v.at[jj], sem))
            lanes = uvec & 127
            for jj, c in enumerate(copies):
                c.wait()
                vals = plsc.load_gather(
                    col_v,
                    [jnp.full((_L,), jj, jnp.int32), iota,
                     jnp.full((_L,), lanes[jj], jnp.int32)])
                out_v[2 * b + (jj >> 3), pl.ds((jj & 7) * _D, _D)] = vals
            return carry

        lax.fori_loop(0, _NB, batch, 0)
        pltpu.sync_copy(out_v, out_hbm.at[wid])

    return body(idx, tbl_t)


def kernel(user, user_table, item_table):
    batch = user.shape[0]
    rows = _gather(user, user_table.T)
    return (rows.reshape(batch, _D), item_table)
